# Initial kernel scaffold; baseline (speedup 1.0000x reference)
import jax
import jax.numpy as jnp
from jax.experimental import pallas as pl
from jax.experimental.pallas import tpu as pltpu

_WIN = 32


def _body(pos_ref, img_ref, w_ref, out_ref):
    i = pl.program_id(0)
    p0 = pos_ref[i, 0]
    p1 = pos_ref[i, 1]
    out_ref[...] = img_ref[...]
    out_ref[0, pl.ds(p0, _WIN), pl.ds(p1, _WIN)] = w_ref[...]


def kernel(images, position, W):
    B, H, Wimg = images.shape
    grid_spec = pltpu.PrefetchScalarGridSpec(
        num_scalar_prefetch=1,
        grid=(B,),
        in_specs=[
            pl.BlockSpec((1, H, Wimg), lambda i, pos: (i, 0, 0)),
            pl.BlockSpec((_WIN, _WIN), lambda i, pos: (0, 0)),
        ],
        out_specs=pl.BlockSpec((1, H, Wimg), lambda i, pos: (i, 0, 0)),
    )
    return pl.pallas_call(
        _body,
        grid_spec=grid_spec,
        out_shape=jax.ShapeDtypeStruct(images.shape, images.dtype),
    )(position.astype(jnp.int32), images, W)


# speedup vs baseline: 6.5869x; 6.5869x over previous
import jax
import jax.numpy as jnp
from jax.experimental import pallas as pl
from jax.experimental.pallas import tpu as pltpu

_WIN = 32


def _body(pos_ref, img_ref, wpad_ref, out_ref):
    i = pl.program_id(0)
    p0 = pos_ref[i, 0]
    p1 = pos_ref[i, 1]
    x = img_ref[0]
    H, Wimg = x.shape
    ri = jax.lax.broadcasted_iota(jnp.int32, (H, Wimg), 0)
    ci = jax.lax.broadcasted_iota(jnp.int32, (H, Wimg), 1)
    mask = (ri >= p0) & (ri < p0 + _WIN) & (ci >= p1) & (ci < p1 + _WIN)
    w_shift = pltpu.roll(pltpu.roll(wpad_ref[...], p0, 0), p1, 1)
    out_ref[0] = jnp.where(mask, w_shift, x)


def kernel(images, position, W):
    B, H, Wimg = images.shape
    wpad = jnp.zeros((H, Wimg), dtype=W.dtype).at[:_WIN, :_WIN].set(W)
    grid_spec = pltpu.PrefetchScalarGridSpec(
        num_scalar_prefetch=1,
        grid=(B,),
        in_specs=[
            pl.BlockSpec((1, H, Wimg), lambda i, pos: (i, 0, 0)),
            pl.BlockSpec((H, Wimg), lambda i, pos: (0, 0)),
        ],
        out_specs=pl.BlockSpec((1, H, Wimg), lambda i, pos: (i, 0, 0)),
    )
    return pl.pallas_call(
        _body,
        grid_spec=grid_spec,
        out_shape=jax.ShapeDtypeStruct(images.shape, images.dtype),
    )(position.astype(jnp.int32), images, wpad)


# speedup vs baseline: 8.7402x; 1.3269x over previous
import jax
import jax.numpy as jnp
from jax.experimental import pallas as pl
from jax.experimental.pallas import tpu as pltpu

_WIN = 32
_SLAB = _WIN + 8


def _body(pos_ref, img_ref, wpad_ref, out_ref):
    i = pl.program_id(0)
    p0 = pos_ref[i, 0]
    p1 = pos_ref[i, 1]
    a = pl.multiple_of((p0 // 8) * 8, 8)
    r = p0 - a
    out_ref[...] = img_ref[...]
    slab = img_ref[0, pl.ds(a, _SLAB), :]
    Wimg = slab.shape[1]
    ri = jax.lax.broadcasted_iota(jnp.int32, (_SLAB, Wimg), 0)
    ci = jax.lax.broadcasted_iota(jnp.int32, (_SLAB, Wimg), 1)
    mask = (ri >= r) & (ri < r + _WIN) & (ci >= p1) & (ci < p1 + _WIN)
    w_shift = pltpu.roll(pltpu.roll(wpad_ref[...], r, 0), p1, 1)
    out_ref[0, pl.ds(a, _SLAB), :] = jnp.where(mask, w_shift, slab)


def kernel(images, position, W):
    B, H, Wimg = images.shape
    wpad = jnp.zeros((_SLAB, Wimg), dtype=W.dtype).at[:_WIN, :_WIN].set(W)
    grid_spec = pltpu.PrefetchScalarGridSpec(
        num_scalar_prefetch=1,
        grid=(B,),
        in_specs=[
            pl.BlockSpec((1, H, Wimg), lambda i, pos: (i, 0, 0)),
            pl.BlockSpec((_SLAB, Wimg), lambda i, pos: (0, 0)),
        ],
        out_specs=pl.BlockSpec((1, H, Wimg), lambda i, pos: (i, 0, 0)),
    )
    return pl.pallas_call(
        _body,
        grid_spec=grid_spec,
        out_shape=jax.ShapeDtypeStruct(images.shape, images.dtype),
    )(position.astype(jnp.int32), images, wpad)


# speedup vs baseline: 12.2633x; 1.4031x over previous
import jax
import jax.numpy as jnp
from jax.experimental import pallas as pl
from jax.experimental.pallas import tpu as pltpu

_WIN = 32
_SLAB = _WIN + 8


_BI = 2


def _body(pos_ref, img_ref, wpad_ref, out_ref):
    i = pl.program_id(0)
    out_ref[...] = img_ref[...]
    for j in range(_BI):
        p0 = pos_ref[i * _BI + j, 0]
        p1 = pos_ref[i * _BI + j, 1]
        a = pl.multiple_of((p0 // 8) * 8, 8)
        r = p0 - a
        slab = img_ref[j, pl.ds(a, _SLAB), :]
        Wimg = slab.shape[1]
        ri = jax.lax.broadcasted_iota(jnp.int32, (_SLAB, Wimg), 0)
        ci = jax.lax.broadcasted_iota(jnp.int32, (_SLAB, Wimg), 1)
        mask = (ri >= r) & (ri < r + _WIN) & (ci >= p1) & (ci < p1 + _WIN)
        w_shift = pltpu.roll(pltpu.roll(wpad_ref[...], r, 0), p1, 1)
        out_ref[j, pl.ds(a, _SLAB), :] = jnp.where(mask, w_shift, slab)


def kernel(images, position, W):
    B, H, Wimg = images.shape
    wpad = jnp.zeros((_SLAB, Wimg), dtype=W.dtype).at[:_WIN, :_WIN].set(W)
    grid_spec = pltpu.PrefetchScalarGridSpec(
        num_scalar_prefetch=1,
        grid=(B // _BI,),
        in_specs=[
            pl.BlockSpec((_BI, H, Wimg), lambda i, pos: (i, 0, 0)),
            pl.BlockSpec((_SLAB, Wimg), lambda i, pos: (0, 0)),
        ],
        out_specs=pl.BlockSpec((_BI, H, Wimg), lambda i, pos: (i, 0, 0)),
    )
    return pl.pallas_call(
        _body,
        grid_spec=grid_spec,
        out_shape=jax.ShapeDtypeStruct(images.shape, images.dtype),
    )(position.astype(jnp.int32), images, wpad)


# speedup vs baseline: 13.4216x; 1.0945x over previous
import jax
import jax.numpy as jnp
from jax.experimental import pallas as pl
from jax.experimental.pallas import tpu as pltpu

_WIN = 32
_SLAB = _WIN + 8


_BI = 4


def _body(pos_ref, img_ref, wpad_ref, out_ref):
    i = pl.program_id(0)
    out_ref[...] = img_ref[...]
    for j in range(_BI):
        p0 = pos_ref[i * _BI + j, 0]
        p1 = pos_ref[i * _BI + j, 1]
        a = pl.multiple_of((p0 // 8) * 8, 8)
        r = p0 - a
        slab = img_ref[j, pl.ds(a, _SLAB), :]
        Wimg = slab.shape[1]
        ri = jax.lax.broadcasted_iota(jnp.int32, (_SLAB, Wimg), 0)
        ci = jax.lax.broadcasted_iota(jnp.int32, (_SLAB, Wimg), 1)
        mask = (ri >= r) & (ri < r + _WIN) & (ci >= p1) & (ci < p1 + _WIN)
        w_shift = pltpu.roll(pltpu.roll(wpad_ref[...], r, 0), p1, 1)
        out_ref[j, pl.ds(a, _SLAB), :] = jnp.where(mask, w_shift, slab)


def kernel(images, position, W):
    B, H, Wimg = images.shape
    wpad = jnp.zeros((_SLAB, Wimg), dtype=W.dtype).at[:_WIN, :_WIN].set(W)
    grid_spec = pltpu.PrefetchScalarGridSpec(
        num_scalar_prefetch=1,
        grid=(B // _BI,),
        in_specs=[
            pl.BlockSpec((_BI, H, Wimg), lambda i, pos: (i, 0, 0)),
            pl.BlockSpec((_SLAB, Wimg), lambda i, pos: (0, 0)),
        ],
        out_specs=pl.BlockSpec((_BI, H, Wimg), lambda i, pos: (i, 0, 0)),
    )
    return pl.pallas_call(
        _body,
        grid_spec=grid_spec,
        out_shape=jax.ShapeDtypeStruct(images.shape, images.dtype),
    )(position.astype(jnp.int32), images, wpad)


# speedup vs baseline: 13.5941x; 1.0129x over previous
import jax
import jax.numpy as jnp
from jax.experimental import pallas as pl
from jax.experimental.pallas import tpu as pltpu

_WIN = 32
_SLAB = _WIN + 8


_BI = 8


def _body(pos_ref, img_ref, wpad_ref, out_ref):
    i = pl.program_id(0)
    out_ref[...] = img_ref[...]
    for j in range(_BI):
        p0 = pos_ref[i * _BI + j, 0]
        p1 = pos_ref[i * _BI + j, 1]
        a = pl.multiple_of((p0 // 8) * 8, 8)
        r = p0 - a
        slab = img_ref[j, pl.ds(a, _SLAB), :]
        Wimg = slab.shape[1]
        ri = jax.lax.broadcasted_iota(jnp.int32, (_SLAB, Wimg), 0)
        ci = jax.lax.broadcasted_iota(jnp.int32, (_SLAB, Wimg), 1)
        mask = (ri >= r) & (ri < r + _WIN) & (ci >= p1) & (ci < p1 + _WIN)
        w_shift = pltpu.roll(pltpu.roll(wpad_ref[...], r, 0), p1, 1)
        out_ref[j, pl.ds(a, _SLAB), :] = jnp.where(mask, w_shift, slab)


def kernel(images, position, W):
    B, H, Wimg = images.shape
    wpad = jnp.zeros((_SLAB, Wimg), dtype=W.dtype).at[:_WIN, :_WIN].set(W)
    grid_spec = pltpu.PrefetchScalarGridSpec(
        num_scalar_prefetch=1,
        grid=(B // _BI,),
        in_specs=[
            pl.BlockSpec((_BI, H, Wimg), lambda i, pos: (i, 0, 0)),
            pl.BlockSpec((_SLAB, Wimg), lambda i, pos: (0, 0)),
        ],
        out_specs=pl.BlockSpec((_BI, H, Wimg), lambda i, pos: (i, 0, 0)),
    )
    return pl.pallas_call(
        _body,
        grid_spec=grid_spec,
        out_shape=jax.ShapeDtypeStruct(images.shape, images.dtype),
    )(position.astype(jnp.int32), images, wpad)
